# baseline (speedup 1.0000x reference)
import jax
import jax.numpy as jnp
from jax import lax
from jax.experimental import pallas as pl
from jax.experimental.pallas import tpu as pltpu
from jax.experimental.pallas import tpu_sc as plsc

N = 10000
E = 320000
G_DIM = 128
HALF = G_DIM // 2
SCALE = G_DIM ** (-0.5)

NC = 2
NS = 16
LANES = 16

BN = 1000
EC = 80
N_ECHUNK = E // EC
CPT = N_ECHUNK // NS
FR = 80
N_FCHUNK = N // FR


def _tc_body(s_ref, w_ref, g_ref, b_ref, qv_ref, k_ref):
    x = s_ref[...]
    mean = jnp.mean(x, axis=1, keepdims=True)
    xc = x - mean
    var = jnp.mean(xc * xc, axis=1, keepdims=True)
    sn = xc / jnp.sqrt(var + 1e-5) * g_ref[...] + b_ref[...]
    y = lax.dot_general(sn, w_ref[...], (((1,), (1,)), ((), ())),
                        preferred_element_type=jnp.float32)
    q = y[:, :G_DIM] * SCALE
    v = y[:, 2 * G_DIM:]
    qv_ref[0, :, :HALF] = q[:, :HALF]
    qv_ref[0, :, HALF:] = v[:, :HALF]
    qv_ref[1, :, :HALF] = q[:, HALF:]
    qv_ref[1, :, HALF:] = v[:, HALF:]
    k_ref[0] = y[:, G_DIM:G_DIM + HALF]
    k_ref[1] = y[:, G_DIM + HALF:2 * G_DIM]


def _project(s, W_qkv, ln_gamma, ln_beta):
    return pl.pallas_call(
        _tc_body,
        grid=(N // BN,),
        in_specs=[
            pl.BlockSpec((BN, G_DIM), lambda i: (i, 0)),
            pl.BlockSpec((3 * G_DIM, G_DIM), lambda i: (0, 0)),
            pl.BlockSpec((G_DIM,), lambda i: (0,)),
            pl.BlockSpec((G_DIM,), lambda i: (0,)),
        ],
        out_specs=[
            pl.BlockSpec((2, BN, G_DIM), lambda i: (0, i, 0)),
            pl.BlockSpec((2, BN, HALF), lambda i: (0, i, 0)),
        ],
        out_shape=[
            jax.ShapeDtypeStruct((2, N, G_DIM), jnp.float32),
            jax.ShapeDtypeStruct((2, N, HALF), jnp.float32),
        ],
    )(s, W_qkv, ln_gamma, ln_beta)


def _sc_body(qv_hbm, k_hbm, ei_hbm, out_hbm,
             ei0, ei1, srca0, srca1, dstk0, dstk1, dstr0, dstr1, dstr2,
             qv0, qv1, kb0, kb1, kb2, wv0, wv1,
             num_sp, den_sp,
             semi0, semi1, semq0, semq1, semk0, semk1, semk2,
             semn0, semn1, semd0, semd1, semd2):
    cid = lax.axis_index("c")
    sid = lax.axis_index("s")
    cn = (cid * N).astype(jnp.int32)
    ei = (ei0, ei1)
    srca = (srca0, srca1)
    dstk = (dstk0, dstk1)
    dstr = (dstr0, dstr1, dstr2)
    qv_rows = (qv0, qv1)
    kb = (kb0, kb1, kb2)
    wv = (wv0, wv1)
    semi = (semi0, semi1)
    semq = (semq0, semq1)
    semk = (semk0, semk1, semk2)
    semn = (semn0, semn1)
    semd = (semd0, semd1, semd2)

    @plsc.parallel_loop(0, FR)
    def _zero_rows(r):
        for j in range(HALF // LANES):
            kb0[r, pl.ds(j * LANES, LANES)] = jnp.zeros((LANES,), jnp.float32)

    @pl.loop(sid, N_FCHUNK, step=NS)
    def _zero_sp(j):
        rbase = pl.multiple_of(j * FR, 8)
        pltpu.sync_copy(kb0, num_sp.at[pl.ds(rbase, FR)])
        pltpu.sync_copy(kb0, den_sp.at[pl.ds(rbase, FR)])

    plsc.subcore_barrier()

    cbase = sid * CPT

    def adjust(p2, p3):
        for g in range(EC // LANES):
            sl = pl.ds(g * LANES, LANES)
            srca[p2][sl] = ei[p2][0, sl] + cn
            d = ei[p2][1, sl]
            dstk[p2][sl] = d + cn
            dstr[p3][sl] = d

    def issue_gathers(p2, p3):
        pltpu.async_copy(qv_hbm.at[srca[p2]], qv_rows[p2], semq[p2])
        pltpu.async_copy(k_hbm.at[dstk[p2]], kb[p3], semk[p3])

    def chunk_step(j, b2, b3, do_drain, do_idx, do_prep):
        nb2 = 1 - b2
        b3n = (b3 + 1) % 3

        if do_drain:
            pltpu.make_async_copy(wv[b2], num_sp.at[dstr[b3n]],
                                  semn[b2]).wait()
            pltpu.make_async_copy(kb[b3n], den_sp.at[dstr[b3n]],
                                  semd[b3n]).wait()
        if do_idx:
            pltpu.async_copy(ei_hbm.at[cbase + j + 2], ei[b2], semi[b2])
        if do_prep:
            pltpu.make_async_copy(ei_hbm.at[cbase + j + 1], ei[nb2],
                                  semi[nb2]).wait()
            adjust(nb2, b3n)
            issue_gathers(nb2, b3n)

        pltpu.make_async_copy(qv_hbm.at[srca[b2]], qv_rows[b2],
                              semq[b2]).wait()
        pltpu.make_async_copy(k_hbm.at[dstk[b2]], kb[b3], semk[b3]).wait()

        qvb = qv_rows[b2]
        kbb = kb[b3]
        wvb = wv[b2]

        @plsc.parallel_loop(0, EC, unroll=8)
        def _edge(e):
            for g in range(HALF // LANES):
                sl = pl.ds(g * LANES, LANES)
                sh = pl.ds(HALF + g * LANES, LANES)
                w = jnp.exp(qvb[e, sl] * kbb[e, sl])
                kbb[e, sl] = w
                wvb[e, sl] = w * qvb[e, sh]

        pltpu.async_copy(wvb, num_sp.at[dstr[b3]], semn[b2], add=True)
        pltpu.async_copy(kbb, den_sp.at[dstr[b3]], semd[b3], add=True)

    pltpu.async_copy(ei_hbm.at[cbase], ei0, semi0)
    pltpu.async_copy(ei_hbm.at[cbase + 1], ei1, semi1)
    pltpu.make_async_copy(ei_hbm.at[cbase], ei0, semi0).wait()
    adjust(0, 0)
    issue_gathers(0, 0)

    for j in range(6):
        chunk_step(j, j % 2, j % 3, j >= 2, True, True)

    @pl.loop(6, CPT - 4, step=6)
    def _steady(o):
        for t in range(6):
            chunk_step(o + t, t % 2, t % 3, True, True, True)

    for j in range(CPT - 4, CPT):
        chunk_step(j, j % 2, j % 3, True, j + 2 < CPT, j + 1 < CPT)

    for j in range(CPT - 2, CPT):
        pltpu.make_async_copy(wv[j % 2], num_sp.at[dstr[j % 3]],
                              semn[j % 2]).wait()
        pltpu.make_async_copy(kb[j % 3], den_sp.at[dstr[j % 3]],
                              semd[j % 3]).wait()

    plsc.subcore_barrier()

    @pl.loop(sid, N_FCHUNK, step=NS)
    def _fin_chunk(j):
        rbase = pl.multiple_of(j * FR, 8)
        pltpu.sync_copy(num_sp.at[pl.ds(rbase, FR)], kb0)
        pltpu.sync_copy(den_sp.at[pl.ds(rbase, FR)], kb1)

        @plsc.parallel_loop(0, FR, unroll=2)
        def _fin_row(r):
            for g in range(HALF // LANES):
                sl = pl.ds(g * LANES, LANES)
                num = kb0[r, sl]
                den = kb1[r, sl]
                wv0[r, sl] = jnp.where(den != 0.0, num / den, 0.0)

        pltpu.sync_copy(wv0, out_hbm.at[cid, pl.ds(rbase, FR)])


_sc_call = pl.kernel(
    _sc_body,
    out_type=jax.ShapeDtypeStruct((2, N, HALF), jnp.float32),
    mesh=plsc.VectorSubcoreMesh(core_axis_name="c", subcore_axis_name="s",
                                num_cores=NC, num_subcores=NS),
    scratch_types=[
        pltpu.VMEM((2, EC), jnp.int32),
        pltpu.VMEM((2, EC), jnp.int32),
        pltpu.VMEM((EC,), jnp.int32),
        pltpu.VMEM((EC,), jnp.int32),
        pltpu.VMEM((EC,), jnp.int32),
        pltpu.VMEM((EC,), jnp.int32),
        pltpu.VMEM((EC,), jnp.int32),
        pltpu.VMEM((EC,), jnp.int32),
        pltpu.VMEM((EC,), jnp.int32),
        pltpu.VMEM((EC, G_DIM), jnp.float32),
        pltpu.VMEM((EC, G_DIM), jnp.float32),
        pltpu.VMEM((EC, HALF), jnp.float32),
        pltpu.VMEM((EC, HALF), jnp.float32),
        pltpu.VMEM((EC, HALF), jnp.float32),
        pltpu.VMEM((EC, HALF), jnp.float32),
        pltpu.VMEM((EC, HALF), jnp.float32),
        pltpu.VMEM_SHARED((N, HALF), jnp.float32),
        pltpu.VMEM_SHARED((N, HALF), jnp.float32),
        pltpu.SemaphoreType.DMA,
        pltpu.SemaphoreType.DMA,
        pltpu.SemaphoreType.DMA,
        pltpu.SemaphoreType.DMA,
        pltpu.SemaphoreType.DMA,
        pltpu.SemaphoreType.DMA,
        pltpu.SemaphoreType.DMA,
        pltpu.SemaphoreType.DMA,
        pltpu.SemaphoreType.DMA,
        pltpu.SemaphoreType.DMA,
        pltpu.SemaphoreType.DMA,
        pltpu.SemaphoreType.DMA,
    ],
    compiler_params=pltpu.CompilerParams(use_tc_tiling_on_sc=False),
)


def kernel(s, edge_index, W_qkv, ln_gamma, ln_beta):
    qv3, k3 = _project(s, W_qkv, ln_gamma, ln_beta)
    qv = qv3.reshape(2 * N, G_DIM)
    kt = k3.reshape(2 * N, HALF)
    ei = edge_index.reshape(2, N_ECHUNK, EC).transpose(1, 0, 2)
    out3 = _sc_call(qv, kt, ei)
    return out3.transpose(1, 0, 2).reshape(N, G_DIM)


# speedup vs baseline: 1.0888x; 1.0888x over previous
import jax
import jax.numpy as jnp
from jax import lax
from jax.experimental import pallas as pl
from jax.experimental.pallas import tpu as pltpu
from jax.experimental.pallas import tpu_sc as plsc

N = 10000
E = 320000
G_DIM = 128
HALF = G_DIM // 2
SCALE = G_DIM ** (-0.5)

NC = 2
NS = 16
LANES = 16

BN = 1000
EC = 80
N_ECHUNK = E // EC
CPT = N_ECHUNK // NS
FR = 80
N_FCHUNK = N // FR


def _tc_body(s_ref, w_ref, g_ref, b_ref, qv_ref, k_ref):
    x = s_ref[...]
    mean = jnp.mean(x, axis=1, keepdims=True)
    xc = x - mean
    var = jnp.mean(xc * xc, axis=1, keepdims=True)
    sn = xc / jnp.sqrt(var + 1e-5) * g_ref[...] + b_ref[...]
    y = lax.dot_general(sn, w_ref[...], (((1,), (1,)), ((), ())),
                        preferred_element_type=jnp.float32)
    q = y[:, :G_DIM] * SCALE
    v = y[:, 2 * G_DIM:]
    qv_ref[0, :, :HALF] = q[:, :HALF]
    qv_ref[0, :, HALF:] = v[:, :HALF]
    qv_ref[1, :, :HALF] = q[:, HALF:]
    qv_ref[1, :, HALF:] = v[:, HALF:]
    k_ref[0] = y[:, G_DIM:G_DIM + HALF]
    k_ref[1] = y[:, G_DIM + HALF:2 * G_DIM]


def _project(s, W_qkv, ln_gamma, ln_beta):
    return pl.pallas_call(
        _tc_body,
        grid=(N // BN,),
        in_specs=[
            pl.BlockSpec((BN, G_DIM), lambda i: (i, 0)),
            pl.BlockSpec((3 * G_DIM, G_DIM), lambda i: (0, 0)),
            pl.BlockSpec((G_DIM,), lambda i: (0,)),
            pl.BlockSpec((G_DIM,), lambda i: (0,)),
        ],
        out_specs=[
            pl.BlockSpec((2, BN, G_DIM), lambda i: (0, i, 0)),
            pl.BlockSpec((2, BN, HALF), lambda i: (0, i, 0)),
        ],
        out_shape=[
            jax.ShapeDtypeStruct((2, N, G_DIM), jnp.float32),
            jax.ShapeDtypeStruct((2, N, HALF), jnp.float32),
        ],
    )(s, W_qkv, ln_gamma, ln_beta)


def _sc_body(qv_hbm, k_hbm, ei_hbm, out_hbm,
             ei0, ei1, srca0, srca1, dstk0, dstk1, dstr0, dstr1, dstr2,
             qv0, qv1, kb0, kb1, kb2, wv0, wv1,
             num_sp, den_sp,
             semi0, semi1, semq0, semq1, semk0, semk1, semk2,
             semn0, semn1, semd0, semd1, semd2):
    cid = lax.axis_index("c")
    sid = lax.axis_index("s")
    cn = (cid * N).astype(jnp.int32)
    ei = (ei0, ei1)
    srca = (srca0, srca1)
    dstk = (dstk0, dstk1)
    dstr = (dstr0, dstr1, dstr2)
    qv_rows = (qv0, qv1)
    kb = (kb0, kb1, kb2)
    wv = (wv0, wv1)
    semi = (semi0, semi1)
    semq = (semq0, semq1)
    semk = (semk0, semk1, semk2)
    semn = (semn0, semn1)
    semd = (semd0, semd1, semd2)

    @plsc.parallel_loop(0, FR)
    def _zero_rows(r):
        for j in range(HALF // LANES):
            kb0[r, pl.ds(j * LANES, LANES)] = jnp.zeros((LANES,), jnp.float32)

    @pl.loop(sid, N_FCHUNK, step=NS)
    def _zero_sp(j):
        rbase = pl.multiple_of(j * FR, 8)
        pltpu.sync_copy(kb0, num_sp.at[pl.ds(rbase, FR)])
        pltpu.sync_copy(kb0, den_sp.at[pl.ds(rbase, FR)])

    plsc.subcore_barrier()

    cbase = sid * CPT

    def adjust(p2, p3):
        for g in range(EC // LANES):
            sl = pl.ds(g * LANES, LANES)
            srca[p2][sl] = ei[p2][0, sl] + cn
            d = ei[p2][1, sl]
            dstk[p2][sl] = d + cn
            dstr[p3][sl] = d

    def issue_gathers(p2, p3):
        pltpu.async_copy(qv_hbm.at[srca[p2]], qv_rows[p2], semq[p2])
        pltpu.async_copy(k_hbm.at[dstk[p2]], kb[p3], semk[p3])

    def chunk_step(j, b2, b3, do_drain, do_idx, do_prep):
        nb2 = 1 - b2
        b3n = (b3 + 1) % 3

        if do_drain:
            pltpu.make_async_copy(wv[b2], num_sp.at[dstr[b3n]],
                                  semn[b2]).wait()
            pltpu.make_async_copy(kb[b3n], den_sp.at[dstr[b3n]],
                                  semd[b3n]).wait()
        if do_idx:
            pltpu.async_copy(ei_hbm.at[cbase + j + 2], ei[b2], semi[b2])
        if do_prep:
            pltpu.make_async_copy(ei_hbm.at[cbase + j + 1], ei[nb2],
                                  semi[nb2]).wait()
            adjust(nb2, b3n)
            issue_gathers(nb2, b3n)

        pltpu.make_async_copy(qv_hbm.at[srca[b2]], qv_rows[b2],
                              semq[b2]).wait()
        pltpu.make_async_copy(k_hbm.at[dstk[b2]], kb[b3], semk[b3]).wait()

        qvb = qv_rows[b2]
        kbb = kb[b3]
        wvb = wv[b2]

        @plsc.parallel_loop(0, EC, unroll=4)
        def _edge(e):
            for g in range(HALF // LANES):
                sl = pl.ds(g * LANES, LANES)
                sh = pl.ds(HALF + g * LANES, LANES)
                w = jnp.exp(qvb[e, sl] * kbb[e, sl])
                kbb[e, sl] = w
                wvb[e, sl] = w * qvb[e, sh]

        pltpu.async_copy(wvb, num_sp.at[dstr[b3]], semn[b2], add=True)
        pltpu.async_copy(kbb, den_sp.at[dstr[b3]], semd[b3], add=True)

    pltpu.async_copy(ei_hbm.at[cbase], ei0, semi0)
    pltpu.async_copy(ei_hbm.at[cbase + 1], ei1, semi1)
    pltpu.make_async_copy(ei_hbm.at[cbase], ei0, semi0).wait()
    adjust(0, 0)
    issue_gathers(0, 0)

    for j in range(6):
        chunk_step(j, j % 2, j % 3, j >= 2, True, True)

    @pl.loop(6, CPT - 4, step=6)
    def _steady(o):
        for t in range(6):
            chunk_step(o + t, t % 2, t % 3, True, True, True)

    for j in range(CPT - 4, CPT):
        chunk_step(j, j % 2, j % 3, True, j + 2 < CPT, j + 1 < CPT)

    for j in range(CPT - 2, CPT):
        pltpu.make_async_copy(wv[j % 2], num_sp.at[dstr[j % 3]],
                              semn[j % 2]).wait()
        pltpu.make_async_copy(kb[j % 3], den_sp.at[dstr[j % 3]],
                              semd[j % 3]).wait()

    plsc.subcore_barrier()

    @pl.loop(sid, N_FCHUNK, step=NS)
    def _fin_chunk(j):
        rbase = pl.multiple_of(j * FR, 8)
        pltpu.sync_copy(num_sp.at[pl.ds(rbase, FR)], kb0)
        pltpu.sync_copy(den_sp.at[pl.ds(rbase, FR)], kb1)

        @plsc.parallel_loop(0, FR, unroll=2)
        def _fin_row(r):
            for g in range(HALF // LANES):
                sl = pl.ds(g * LANES, LANES)
                num = kb0[r, sl]
                den = kb1[r, sl]
                wv0[r, sl] = jnp.where(den != 0.0, num / den, 0.0)

        pltpu.sync_copy(wv0, out_hbm.at[pl.ds(rbase, FR),
                                        pl.ds(cid * HALF, HALF)])


_sc_call = pl.kernel(
    _sc_body,
    out_type=jax.ShapeDtypeStruct((N, G_DIM), jnp.float32),
    mesh=plsc.VectorSubcoreMesh(core_axis_name="c", subcore_axis_name="s",
                                num_cores=NC, num_subcores=NS),
    scratch_types=[
        pltpu.VMEM((2, EC), jnp.int32),
        pltpu.VMEM((2, EC), jnp.int32),
        pltpu.VMEM((EC,), jnp.int32),
        pltpu.VMEM((EC,), jnp.int32),
        pltpu.VMEM((EC,), jnp.int32),
        pltpu.VMEM((EC,), jnp.int32),
        pltpu.VMEM((EC,), jnp.int32),
        pltpu.VMEM((EC,), jnp.int32),
        pltpu.VMEM((EC,), jnp.int32),
        pltpu.VMEM((EC, G_DIM), jnp.float32),
        pltpu.VMEM((EC, G_DIM), jnp.float32),
        pltpu.VMEM((EC, HALF), jnp.float32),
        pltpu.VMEM((EC, HALF), jnp.float32),
        pltpu.VMEM((EC, HALF), jnp.float32),
        pltpu.VMEM((EC, HALF), jnp.float32),
        pltpu.VMEM((EC, HALF), jnp.float32),
        pltpu.VMEM_SHARED((N, HALF), jnp.float32),
        pltpu.VMEM_SHARED((N, HALF), jnp.float32),
        pltpu.SemaphoreType.DMA,
        pltpu.SemaphoreType.DMA,
        pltpu.SemaphoreType.DMA,
        pltpu.SemaphoreType.DMA,
        pltpu.SemaphoreType.DMA,
        pltpu.SemaphoreType.DMA,
        pltpu.SemaphoreType.DMA,
        pltpu.SemaphoreType.DMA,
        pltpu.SemaphoreType.DMA,
        pltpu.SemaphoreType.DMA,
        pltpu.SemaphoreType.DMA,
        pltpu.SemaphoreType.DMA,
    ],
    compiler_params=pltpu.CompilerParams(use_tc_tiling_on_sc=False),
)


def kernel(s, edge_index, W_qkv, ln_gamma, ln_beta):
    qv3, k3 = _project(s, W_qkv, ln_gamma, ln_beta)
    qv = qv3.reshape(2 * N, G_DIM)
    kt = k3.reshape(2 * N, HALF)
    ei = edge_index.reshape(2, N_ECHUNK, EC).transpose(1, 0, 2)
    return _sc_call(qv, kt, ei)


# speedup vs baseline: 1.0931x; 1.0040x over previous
import jax
import jax.numpy as jnp
from jax import lax
from jax.experimental import pallas as pl
from jax.experimental.pallas import tpu as pltpu
from jax.experimental.pallas import tpu_sc as plsc

N = 10000
E = 320000
G_DIM = 128
HALF = G_DIM // 2
SCALE = G_DIM ** (-0.5)

NC = 2
NS = 16
LANES = 16

BN = 1000
EC = 80
N_ECHUNK = E // EC
CPT = N_ECHUNK // NS
FR = 80
N_FCHUNK = N // FR


def _tc_body(s_ref, w_ref, g_ref, b_ref, qv_ref, k_ref):
    x = s_ref[...]
    mean = jnp.mean(x, axis=1, keepdims=True)
    xc = x - mean
    var = jnp.mean(xc * xc, axis=1, keepdims=True)
    sn = xc / jnp.sqrt(var + 1e-5) * g_ref[...] + b_ref[...]
    y = lax.dot_general(sn, w_ref[...], (((1,), (1,)), ((), ())),
                        preferred_element_type=jnp.float32)
    q = y[:, :G_DIM] * SCALE
    v = y[:, 2 * G_DIM:]
    qv_ref[0, :, :HALF] = q[:, :HALF]
    qv_ref[0, :, HALF:] = v[:, :HALF]
    qv_ref[1, :, :HALF] = q[:, HALF:]
    qv_ref[1, :, HALF:] = v[:, HALF:]
    k_ref[0] = y[:, G_DIM:G_DIM + HALF]
    k_ref[1] = y[:, G_DIM + HALF:2 * G_DIM]


def _project(s, W_qkv, ln_gamma, ln_beta):
    return pl.pallas_call(
        _tc_body,
        grid=(N // BN,),
        in_specs=[
            pl.BlockSpec((BN, G_DIM), lambda i: (i, 0)),
            pl.BlockSpec((3 * G_DIM, G_DIM), lambda i: (0, 0)),
            pl.BlockSpec((G_DIM,), lambda i: (0,)),
            pl.BlockSpec((G_DIM,), lambda i: (0,)),
        ],
        out_specs=[
            pl.BlockSpec((2, BN, G_DIM), lambda i: (0, i, 0)),
            pl.BlockSpec((2, BN, HALF), lambda i: (0, i, 0)),
        ],
        out_shape=[
            jax.ShapeDtypeStruct((2, N, G_DIM), jnp.float32),
            jax.ShapeDtypeStruct((2, N, HALF), jnp.float32),
        ],
    )(s, W_qkv, ln_gamma, ln_beta)


def _sc_body(qv_hbm, k_hbm, ei_hbm, out_hbm,
             ei0, ei1, srca0, srca1, dstk0, dstk1, dstr0, dstr1, dstr2,
             qv0, qv1, kb0, kb1, kb2, wv0, wv1,
             num_sp, den_sp,
             semi0, semi1, semq0, semq1, semk0, semk1, semk2,
             semn0, semn1, semd0, semd1, semd2):
    cid = lax.axis_index("c")
    sid = lax.axis_index("s")
    cn = (cid * N).astype(jnp.int32)
    ei = (ei0, ei1)
    srca = (srca0, srca1)
    dstk = (dstk0, dstk1)
    dstr = (dstr0, dstr1, dstr2)
    qv_rows = (qv0, qv1)
    kb = (kb0, kb1, kb2)
    wv = (wv0, wv1)
    semi = (semi0, semi1)
    semq = (semq0, semq1)
    semk = (semk0, semk1, semk2)
    semn = (semn0, semn1)
    semd = (semd0, semd1, semd2)

    @plsc.parallel_loop(0, FR)
    def _zero_rows(r):
        for j in range(HALF // LANES):
            kb0[r, pl.ds(j * LANES, LANES)] = jnp.zeros((LANES,), jnp.float32)

    @pl.loop(sid, N_FCHUNK, step=NS)
    def _zero_sp(j):
        rbase = pl.multiple_of(j * FR, 8)
        pltpu.async_copy(kb0, num_sp.at[pl.ds(rbase, FR)], semq0)
        pltpu.async_copy(kb0, den_sp.at[pl.ds(rbase, FR)], semk0)

    @pl.loop(sid, N_FCHUNK, step=NS)
    def _zero_drain(j):
        rbase = pl.multiple_of(j * FR, 8)
        pltpu.make_async_copy(kb0, num_sp.at[pl.ds(rbase, FR)], semq0).wait()
        pltpu.make_async_copy(kb0, den_sp.at[pl.ds(rbase, FR)], semk0).wait()

    plsc.subcore_barrier()

    cbase = sid * CPT

    def adjust(p2, p3):
        for g in range(EC // LANES):
            sl = pl.ds(g * LANES, LANES)
            srca[p2][sl] = ei[p2][0, sl] + cn
            d = ei[p2][1, sl]
            dstk[p2][sl] = d + cn
            dstr[p3][sl] = d

    def issue_gathers(p2, p3):
        pltpu.async_copy(qv_hbm.at[srca[p2]], qv_rows[p2], semq[p2])
        pltpu.async_copy(k_hbm.at[dstk[p2]], kb[p3], semk[p3])

    def chunk_step(j, b2, b3, do_drain, do_idx, do_prep):
        nb2 = 1 - b2
        b3n = (b3 + 1) % 3

        if do_drain:
            pltpu.make_async_copy(wv[b2], num_sp.at[dstr[b3n]],
                                  semn[b2]).wait()
            pltpu.make_async_copy(kb[b3n], den_sp.at[dstr[b3n]],
                                  semd[b3n]).wait()
        if do_idx:
            pltpu.async_copy(ei_hbm.at[cbase + j + 2], ei[b2], semi[b2])
        if do_prep:
            pltpu.make_async_copy(ei_hbm.at[cbase + j + 1], ei[nb2],
                                  semi[nb2]).wait()
            adjust(nb2, b3n)
            issue_gathers(nb2, b3n)

        pltpu.make_async_copy(qv_hbm.at[srca[b2]], qv_rows[b2],
                              semq[b2]).wait()
        pltpu.make_async_copy(k_hbm.at[dstk[b2]], kb[b3], semk[b3]).wait()

        qvb = qv_rows[b2]
        kbb = kb[b3]
        wvb = wv[b2]

        @plsc.parallel_loop(0, EC, unroll=4)
        def _edge(e):
            for g in range(HALF // LANES):
                sl = pl.ds(g * LANES, LANES)
                sh = pl.ds(HALF + g * LANES, LANES)
                w = jnp.exp(qvb[e, sl] * kbb[e, sl])
                kbb[e, sl] = w
                wvb[e, sl] = w * qvb[e, sh]

        pltpu.async_copy(wvb, num_sp.at[dstr[b3]], semn[b2], add=True)
        pltpu.async_copy(kbb, den_sp.at[dstr[b3]], semd[b3], add=True)

    pltpu.async_copy(ei_hbm.at[cbase], ei0, semi0)
    pltpu.async_copy(ei_hbm.at[cbase + 1], ei1, semi1)
    pltpu.make_async_copy(ei_hbm.at[cbase], ei0, semi0).wait()
    adjust(0, 0)
    issue_gathers(0, 0)

    for j in range(6):
        chunk_step(j, j % 2, j % 3, j >= 2, True, True)

    @pl.loop(6, CPT - 4, step=6)
    def _steady(o):
        for t in range(6):
            chunk_step(o + t, t % 2, t % 3, True, True, True)

    for j in range(CPT - 4, CPT):
        chunk_step(j, j % 2, j % 3, True, j + 2 < CPT, j + 1 < CPT)

    for j in range(CPT - 2, CPT):
        pltpu.make_async_copy(wv[j % 2], num_sp.at[dstr[j % 3]],
                              semn[j % 2]).wait()
        pltpu.make_async_copy(kb[j % 3], den_sp.at[dstr[j % 3]],
                              semd[j % 3]).wait()

    plsc.subcore_barrier()

    def _out_slice(rbase):
        return out_hbm.at[pl.ds(rbase, FR), pl.ds(cid * HALF, HALF)]

    @pl.loop(sid, N_FCHUNK, step=NS)
    def _fin_chunk(j):
        rbase = pl.multiple_of(j * FR, 8)
        pltpu.async_copy(num_sp.at[pl.ds(rbase, FR)], kb0, semq0)
        pltpu.async_copy(den_sp.at[pl.ds(rbase, FR)], kb1, semk0)
        pltpu.make_async_copy(num_sp.at[pl.ds(rbase, FR)], kb0, semq0).wait()
        pltpu.make_async_copy(den_sp.at[pl.ds(rbase, FR)], kb1, semk0).wait()

        @pl.when(j > sid)
        def _():
            pltpu.make_async_copy(wv0, _out_slice(rbase), semn0).wait()

        @plsc.parallel_loop(0, FR, unroll=2)
        def _fin_row(r):
            for g in range(HALF // LANES):
                sl = pl.ds(g * LANES, LANES)
                num = kb0[r, sl]
                den = kb1[r, sl]
                wv0[r, sl] = jnp.where(den != 0.0, num / den, 0.0)

        pltpu.async_copy(wv0, _out_slice(rbase), semn0)

    pltpu.make_async_copy(wv0, _out_slice(pl.multiple_of(sid * FR, 8)),
                          semn0).wait()


_sc_call = pl.kernel(
    _sc_body,
    out_type=jax.ShapeDtypeStruct((N, G_DIM), jnp.float32),
    mesh=plsc.VectorSubcoreMesh(core_axis_name="c", subcore_axis_name="s",
                                num_cores=NC, num_subcores=NS),
    scratch_types=[
        pltpu.VMEM((2, EC), jnp.int32),
        pltpu.VMEM((2, EC), jnp.int32),
        pltpu.VMEM((EC,), jnp.int32),
        pltpu.VMEM((EC,), jnp.int32),
        pltpu.VMEM((EC,), jnp.int32),
        pltpu.VMEM((EC,), jnp.int32),
        pltpu.VMEM((EC,), jnp.int32),
        pltpu.VMEM((EC,), jnp.int32),
        pltpu.VMEM((EC,), jnp.int32),
        pltpu.VMEM((EC, G_DIM), jnp.float32),
        pltpu.VMEM((EC, G_DIM), jnp.float32),
        pltpu.VMEM((EC, HALF), jnp.float32),
        pltpu.VMEM((EC, HALF), jnp.float32),
        pltpu.VMEM((EC, HALF), jnp.float32),
        pltpu.VMEM((EC, HALF), jnp.float32),
        pltpu.VMEM((EC, HALF), jnp.float32),
        pltpu.VMEM_SHARED((N, HALF), jnp.float32),
        pltpu.VMEM_SHARED((N, HALF), jnp.float32),
        pltpu.SemaphoreType.DMA,
        pltpu.SemaphoreType.DMA,
        pltpu.SemaphoreType.DMA,
        pltpu.SemaphoreType.DMA,
        pltpu.SemaphoreType.DMA,
        pltpu.SemaphoreType.DMA,
        pltpu.SemaphoreType.DMA,
        pltpu.SemaphoreType.DMA,
        pltpu.SemaphoreType.DMA,
        pltpu.SemaphoreType.DMA,
        pltpu.SemaphoreType.DMA,
        pltpu.SemaphoreType.DMA,
    ],
    compiler_params=pltpu.CompilerParams(use_tc_tiling_on_sc=False),
)


def kernel(s, edge_index, W_qkv, ln_gamma, ln_beta):
    qv3, k3 = _project(s, W_qkv, ln_gamma, ln_beta)
    qv = qv3.reshape(2 * N, G_DIM)
    kt = k3.reshape(2 * N, HALF)
    ei = edge_index.reshape(2, N_ECHUNK, EC).transpose(1, 0, 2)
    return _sc_call(qv, kt, ei)


# speedup vs baseline: 1.0974x; 1.0039x over previous
import jax
import jax.numpy as jnp
from jax import lax
from jax.experimental import pallas as pl
from jax.experimental.pallas import tpu as pltpu
from jax.experimental.pallas import tpu_sc as plsc

N = 10000
E = 320000
G_DIM = 128
HALF = G_DIM // 2
SCALE = G_DIM ** (-0.5)

NC = 2
NS = 16
LANES = 16

BN = 2000
EC = 80
N_ECHUNK = E // EC
CPT = N_ECHUNK // NS
FR = 80
N_FCHUNK = N // FR


def _tc_body(s_ref, w_ref, g_ref, b_ref, qv_ref, k_ref):
    x = s_ref[...]
    mean = jnp.mean(x, axis=1, keepdims=True)
    xc = x - mean
    var = jnp.mean(xc * xc, axis=1, keepdims=True)
    sn = xc / jnp.sqrt(var + 1e-5) * g_ref[...] + b_ref[...]
    y = lax.dot_general(sn, w_ref[...], (((1,), (1,)), ((), ())),
                        preferred_element_type=jnp.float32)
    q = y[:, :G_DIM] * SCALE
    v = y[:, 2 * G_DIM:]
    qv_ref[0, :, :HALF] = q[:, :HALF]
    qv_ref[0, :, HALF:] = v[:, :HALF]
    qv_ref[1, :, :HALF] = q[:, HALF:]
    qv_ref[1, :, HALF:] = v[:, HALF:]
    k_ref[0] = y[:, G_DIM:G_DIM + HALF]
    k_ref[1] = y[:, G_DIM + HALF:2 * G_DIM]


def _project(s, W_qkv, ln_gamma, ln_beta):
    return pl.pallas_call(
        _tc_body,
        grid=(N // BN,),
        in_specs=[
            pl.BlockSpec((BN, G_DIM), lambda i: (i, 0)),
            pl.BlockSpec((3 * G_DIM, G_DIM), lambda i: (0, 0)),
            pl.BlockSpec((G_DIM,), lambda i: (0,)),
            pl.BlockSpec((G_DIM,), lambda i: (0,)),
        ],
        out_specs=[
            pl.BlockSpec((2, BN, G_DIM), lambda i: (0, i, 0)),
            pl.BlockSpec((2, BN, HALF), lambda i: (0, i, 0)),
        ],
        out_shape=[
            jax.ShapeDtypeStruct((2, N, G_DIM), jnp.float32),
            jax.ShapeDtypeStruct((2, N, HALF), jnp.float32),
        ],
    )(s, W_qkv, ln_gamma, ln_beta)


def _sc_body(qv_hbm, k_hbm, ei_hbm, out_hbm,
             ei0, ei1, srca0, srca1, dstk0, dstk1, dstr0, dstr1, dstr2,
             qv0, qv1, kb0, kb1, kb2, wv0, wv1,
             num_sp, den_sp,
             semi0, semi1, semq0, semq1, semk0, semk1, semk2,
             semn0, semn1, semd0, semd1, semd2):
    cid = lax.axis_index("c")
    sid = lax.axis_index("s")
    cn = (cid * N).astype(jnp.int32)
    ei = (ei0, ei1)
    srca = (srca0, srca1)
    dstk = (dstk0, dstk1)
    dstr = (dstr0, dstr1, dstr2)
    qv_rows = (qv0, qv1)
    kb = (kb0, kb1, kb2)
    wv = (wv0, wv1)
    semi = (semi0, semi1)
    semq = (semq0, semq1)
    semk = (semk0, semk1, semk2)
    semn = (semn0, semn1)
    semd = (semd0, semd1, semd2)

    @plsc.parallel_loop(0, FR)
    def _zero_rows(r):
        for j in range(HALF // LANES):
            kb0[r, pl.ds(j * LANES, LANES)] = jnp.zeros((LANES,), jnp.float32)

    @pl.loop(sid, N_FCHUNK, step=NS)
    def _zero_sp(j):
        rbase = pl.multiple_of(j * FR, 8)
        pltpu.async_copy(kb0, num_sp.at[pl.ds(rbase, FR)], semq0)
        pltpu.async_copy(kb0, den_sp.at[pl.ds(rbase, FR)], semk0)

    @pl.loop(sid, N_FCHUNK, step=NS)
    def _zero_drain(j):
        rbase = pl.multiple_of(j * FR, 8)
        pltpu.make_async_copy(kb0, num_sp.at[pl.ds(rbase, FR)], semq0).wait()
        pltpu.make_async_copy(kb0, den_sp.at[pl.ds(rbase, FR)], semk0).wait()

    plsc.subcore_barrier()

    cbase = sid * CPT

    def adjust(p2, p3):
        for g in range(EC // LANES):
            sl = pl.ds(g * LANES, LANES)
            srca[p2][sl] = ei[p2][0, sl] + cn
            d = ei[p2][1, sl]
            dstk[p2][sl] = d + cn
            dstr[p3][sl] = d

    def issue_gathers(p2, p3):
        pltpu.async_copy(qv_hbm.at[srca[p2]], qv_rows[p2], semq[p2])
        pltpu.async_copy(k_hbm.at[dstk[p2]], kb[p3], semk[p3])

    def chunk_step(j, b2, b3, do_drain, do_idx, do_prep):
        nb2 = 1 - b2
        b3n = (b3 + 1) % 3

        if do_drain:
            pltpu.make_async_copy(wv[b2], num_sp.at[dstr[b3n]],
                                  semn[b2]).wait()
            pltpu.make_async_copy(kb[b3n], den_sp.at[dstr[b3n]],
                                  semd[b3n]).wait()
        if do_idx:
            pltpu.async_copy(ei_hbm.at[cbase + j + 2], ei[b2], semi[b2])
        if do_prep:
            pltpu.make_async_copy(ei_hbm.at[cbase + j + 1], ei[nb2],
                                  semi[nb2]).wait()
            adjust(nb2, b3n)
            issue_gathers(nb2, b3n)

        pltpu.make_async_copy(qv_hbm.at[srca[b2]], qv_rows[b2],
                              semq[b2]).wait()
        pltpu.make_async_copy(k_hbm.at[dstk[b2]], kb[b3], semk[b3]).wait()

        qvb = qv_rows[b2]
        kbb = kb[b3]
        wvb = wv[b2]

        @plsc.parallel_loop(0, EC, unroll=4)
        def _edge(e):
            for g in range(HALF // LANES):
                sl = pl.ds(g * LANES, LANES)
                sh = pl.ds(HALF + g * LANES, LANES)
                w = jnp.exp(qvb[e, sl] * kbb[e, sl])
                kbb[e, sl] = w
                wvb[e, sl] = w * qvb[e, sh]

        pltpu.async_copy(wvb, num_sp.at[dstr[b3]], semn[b2], add=True)
        pltpu.async_copy(kbb, den_sp.at[dstr[b3]], semd[b3], add=True)

    pltpu.async_copy(ei_hbm.at[cbase], ei0, semi0)
    pltpu.async_copy(ei_hbm.at[cbase + 1], ei1, semi1)
    pltpu.make_async_copy(ei_hbm.at[cbase], ei0, semi0).wait()
    adjust(0, 0)
    issue_gathers(0, 0)

    for j in range(6):
        chunk_step(j, j % 2, j % 3, j >= 2, True, True)

    @pl.loop(6, CPT - 4, step=6)
    def _steady(o):
        for t in range(6):
            chunk_step(o + t, t % 2, t % 3, True, True, True)

    for j in range(CPT - 4, CPT):
        chunk_step(j, j % 2, j % 3, True, j + 2 < CPT, j + 1 < CPT)

    for j in range(CPT - 2, CPT):
        pltpu.make_async_copy(wv[j % 2], num_sp.at[dstr[j % 3]],
                              semn[j % 2]).wait()
        pltpu.make_async_copy(kb[j % 3], den_sp.at[dstr[j % 3]],
                              semd[j % 3]).wait()

    plsc.subcore_barrier()

    def _out_slice(rbase):
        return out_hbm.at[pl.ds(rbase, FR), pl.ds(cid * HALF, HALF)]

    @pl.loop(sid, N_FCHUNK, step=NS)
    def _fin_chunk(j):
        rbase = pl.multiple_of(j * FR, 8)
        pltpu.async_copy(num_sp.at[pl.ds(rbase, FR)], kb0, semq0)
        pltpu.async_copy(den_sp.at[pl.ds(rbase, FR)], kb1, semk0)
        pltpu.make_async_copy(num_sp.at[pl.ds(rbase, FR)], kb0, semq0).wait()
        pltpu.make_async_copy(den_sp.at[pl.ds(rbase, FR)], kb1, semk0).wait()

        @pl.when(j > sid)
        def _():
            pltpu.make_async_copy(wv0, _out_slice(rbase), semn0).wait()

        @plsc.parallel_loop(0, FR, unroll=2)
        def _fin_row(r):
            for g in range(HALF // LANES):
                sl = pl.ds(g * LANES, LANES)
                num = kb0[r, sl]
                den = kb1[r, sl]
                wv0[r, sl] = jnp.where(den != 0.0, num / den, 0.0)

        pltpu.async_copy(wv0, _out_slice(rbase), semn0)

    pltpu.make_async_copy(wv0, _out_slice(pl.multiple_of(sid * FR, 8)),
                          semn0).wait()


_sc_call = pl.kernel(
    _sc_body,
    out_type=jax.ShapeDtypeStruct((N, G_DIM), jnp.float32),
    mesh=plsc.VectorSubcoreMesh(core_axis_name="c", subcore_axis_name="s",
                                num_cores=NC, num_subcores=NS),
    scratch_types=[
        pltpu.VMEM((2, EC), jnp.int32),
        pltpu.VMEM((2, EC), jnp.int32),
        pltpu.VMEM((EC,), jnp.int32),
        pltpu.VMEM((EC,), jnp.int32),
        pltpu.VMEM((EC,), jnp.int32),
        pltpu.VMEM((EC,), jnp.int32),
        pltpu.VMEM((EC,), jnp.int32),
        pltpu.VMEM((EC,), jnp.int32),
        pltpu.VMEM((EC,), jnp.int32),
        pltpu.VMEM((EC, G_DIM), jnp.float32),
        pltpu.VMEM((EC, G_DIM), jnp.float32),
        pltpu.VMEM((EC, HALF), jnp.float32),
        pltpu.VMEM((EC, HALF), jnp.float32),
        pltpu.VMEM((EC, HALF), jnp.float32),
        pltpu.VMEM((EC, HALF), jnp.float32),
        pltpu.VMEM((EC, HALF), jnp.float32),
        pltpu.VMEM_SHARED((N, HALF), jnp.float32),
        pltpu.VMEM_SHARED((N, HALF), jnp.float32),
        pltpu.SemaphoreType.DMA,
        pltpu.SemaphoreType.DMA,
        pltpu.SemaphoreType.DMA,
        pltpu.SemaphoreType.DMA,
        pltpu.SemaphoreType.DMA,
        pltpu.SemaphoreType.DMA,
        pltpu.SemaphoreType.DMA,
        pltpu.SemaphoreType.DMA,
        pltpu.SemaphoreType.DMA,
        pltpu.SemaphoreType.DMA,
        pltpu.SemaphoreType.DMA,
        pltpu.SemaphoreType.DMA,
    ],
    compiler_params=pltpu.CompilerParams(use_tc_tiling_on_sc=False),
)


def kernel(s, edge_index, W_qkv, ln_gamma, ln_beta):
    qv3, k3 = _project(s, W_qkv, ln_gamma, ln_beta)
    qv = qv3.reshape(2 * N, G_DIM)
    kt = k3.reshape(2 * N, HALF)
    ei = edge_index.reshape(2, N_ECHUNK, EC).transpose(1, 0, 2)
    return _sc_call(qv, kt, ei)


# speedup vs baseline: 1.1035x; 1.0056x over previous
import jax
import jax.numpy as jnp
from jax import lax
from jax.experimental import pallas as pl
from jax.experimental.pallas import tpu as pltpu
from jax.experimental.pallas import tpu_sc as plsc

N = 10000
E = 320000
G_DIM = 128
HALF = G_DIM // 2
SCALE = G_DIM ** (-0.5)

NC = 2
NS = 16
LANES = 16

BN = 2000
EC = 80
N_ECHUNK = E // EC
CPT = N_ECHUNK // NS
FR = 80
N_FCHUNK = N // FR


def _tc_body(s_ref, w_ref, g_ref, b_ref, qv_ref, k_ref):
    x = s_ref[...]
    mean = jnp.mean(x, axis=1, keepdims=True)
    xc = x - mean
    var = jnp.mean(xc * xc, axis=1, keepdims=True)
    sn = xc / jnp.sqrt(var + 1e-5) * g_ref[...] + b_ref[...]
    y = lax.dot_general(sn, w_ref[...], (((1,), (1,)), ((), ())),
                        preferred_element_type=jnp.float32)
    q = y[:, :G_DIM] * SCALE
    v = y[:, 2 * G_DIM:]
    qv_ref[0, :, :HALF] = q[:, :HALF]
    qv_ref[0, :, HALF:] = v[:, :HALF]
    qv_ref[1, :, :HALF] = q[:, HALF:]
    qv_ref[1, :, HALF:] = v[:, HALF:]
    k_ref[0] = y[:, G_DIM:G_DIM + HALF]
    k_ref[1] = y[:, G_DIM + HALF:2 * G_DIM]


def _project(s, W_qkv, ln_gamma, ln_beta):
    return pl.pallas_call(
        _tc_body,
        grid=(N // BN,),
        in_specs=[
            pl.BlockSpec((BN, G_DIM), lambda i: (i, 0)),
            pl.BlockSpec((3 * G_DIM, G_DIM), lambda i: (0, 0)),
            pl.BlockSpec((G_DIM,), lambda i: (0,)),
            pl.BlockSpec((G_DIM,), lambda i: (0,)),
        ],
        out_specs=[
            pl.BlockSpec((2, BN, G_DIM), lambda i: (0, i, 0)),
            pl.BlockSpec((2, BN, HALF), lambda i: (0, i, 0)),
        ],
        out_shape=[
            jax.ShapeDtypeStruct((2, N, G_DIM), jnp.float32),
            jax.ShapeDtypeStruct((2, N, HALF), jnp.float32),
        ],
    )(s, W_qkv, ln_gamma, ln_beta)


def _sc_body(qv_hbm, k_hbm, ei_hbm, out_hbm,
             ei0, ei1, srca0, srca1, dstk0, dstk1, dstr0, dstr1, dstr2,
             qv0, qv1, kb0, kb1, kb2, wv0, wv1,
             num_sp, den_sp,
             semi0, semi1, semq0, semq1, semr0, semr1, semk0, semk1, semk2,
             semn0, semn1, semd0, semd1, semd2):
    cid = lax.axis_index("c")
    sid = lax.axis_index("s")
    cn = (cid * N).astype(jnp.int32)
    ei = (ei0, ei1)
    srca = (srca0, srca1)
    dstk = (dstk0, dstk1)
    dstr = (dstr0, dstr1, dstr2)
    qv_rows = (qv0, qv1)
    kb = (kb0, kb1, kb2)
    wv = (wv0, wv1)
    semi = (semi0, semi1)
    semq = (semq0, semq1)
    semr = (semr0, semr1)
    semk = (semk0, semk1, semk2)
    semn = (semn0, semn1)
    semd = (semd0, semd1, semd2)

    @plsc.parallel_loop(0, FR)
    def _zero_rows(r):
        for j in range(HALF // LANES):
            kb0[r, pl.ds(j * LANES, LANES)] = jnp.zeros((LANES,), jnp.float32)

    @pl.loop(sid, N_FCHUNK, step=NS)
    def _zero_sp(j):
        rbase = pl.multiple_of(j * FR, 8)
        pltpu.async_copy(kb0, num_sp.at[pl.ds(rbase, FR)], semq0)
        pltpu.async_copy(kb0, den_sp.at[pl.ds(rbase, FR)], semk0)

    @pl.loop(sid, N_FCHUNK, step=NS)
    def _zero_drain(j):
        rbase = pl.multiple_of(j * FR, 8)
        pltpu.make_async_copy(kb0, num_sp.at[pl.ds(rbase, FR)], semq0).wait()
        pltpu.make_async_copy(kb0, den_sp.at[pl.ds(rbase, FR)], semk0).wait()

    plsc.subcore_barrier()

    cbase = sid * CPT

    def adjust(p2, p3):
        for g in range(EC // LANES):
            sl = pl.ds(g * LANES, LANES)
            srca[p2][sl] = ei[p2][0, sl] + cn
            d = ei[p2][1, sl]
            dstk[p2][sl] = d + cn
            dstr[p3][sl] = d

    def issue_gathers(p2, p3):
        h = EC // 2
        pltpu.async_copy(qv_hbm.at[srca[p2].at[pl.ds(0, h)]],
                         qv_rows[p2].at[pl.ds(0, h)], semq[p2])
        pltpu.async_copy(qv_hbm.at[srca[p2].at[pl.ds(h, h)]],
                         qv_rows[p2].at[pl.ds(h, h)], semr[p2])
        pltpu.async_copy(k_hbm.at[dstk[p2]], kb[p3], semk[p3])

    def chunk_step(j, b2, b3, do_drain, do_idx, do_prep):
        nb2 = 1 - b2
        b3n = (b3 + 1) % 3

        if do_drain:
            pltpu.make_async_copy(wv[b2], num_sp.at[dstr[b3n]],
                                  semn[b2]).wait()
            pltpu.make_async_copy(kb[b3n], den_sp.at[dstr[b3n]],
                                  semd[b3n]).wait()
        if do_idx:
            pltpu.async_copy(ei_hbm.at[cbase + j + 2], ei[b2], semi[b2])
        if do_prep:
            pltpu.make_async_copy(ei_hbm.at[cbase + j + 1], ei[nb2],
                                  semi[nb2]).wait()
            adjust(nb2, b3n)
            issue_gathers(nb2, b3n)

        h = EC // 2
        pltpu.make_async_copy(qv_hbm.at[srca[b2].at[pl.ds(0, h)]],
                              qv_rows[b2].at[pl.ds(0, h)], semq[b2]).wait()
        pltpu.make_async_copy(qv_hbm.at[srca[b2].at[pl.ds(h, h)]],
                              qv_rows[b2].at[pl.ds(h, h)], semr[b2]).wait()
        pltpu.make_async_copy(k_hbm.at[dstk[b2]], kb[b3], semk[b3]).wait()

        qvb = qv_rows[b2]
        kbb = kb[b3]
        wvb = wv[b2]

        @plsc.parallel_loop(0, EC, unroll=4)
        def _edge(e):
            for g in range(HALF // LANES):
                sl = pl.ds(g * LANES, LANES)
                sh = pl.ds(HALF + g * LANES, LANES)
                w = jnp.exp(qvb[e, sl] * kbb[e, sl])
                kbb[e, sl] = w
                wvb[e, sl] = w * qvb[e, sh]

        pltpu.async_copy(wvb, num_sp.at[dstr[b3]], semn[b2], add=True)
        pltpu.async_copy(kbb, den_sp.at[dstr[b3]], semd[b3], add=True)

    pltpu.async_copy(ei_hbm.at[cbase], ei0, semi0)
    pltpu.async_copy(ei_hbm.at[cbase + 1], ei1, semi1)
    pltpu.make_async_copy(ei_hbm.at[cbase], ei0, semi0).wait()
    adjust(0, 0)
    issue_gathers(0, 0)

    for j in range(6):
        chunk_step(j, j % 2, j % 3, j >= 2, True, True)

    @pl.loop(6, CPT - 4, step=6)
    def _steady(o):
        for t in range(6):
            chunk_step(o + t, t % 2, t % 3, True, True, True)

    for j in range(CPT - 4, CPT):
        chunk_step(j, j % 2, j % 3, True, j + 2 < CPT, j + 1 < CPT)

    for j in range(CPT - 2, CPT):
        pltpu.make_async_copy(wv[j % 2], num_sp.at[dstr[j % 3]],
                              semn[j % 2]).wait()
        pltpu.make_async_copy(kb[j % 3], den_sp.at[dstr[j % 3]],
                              semd[j % 3]).wait()

    plsc.subcore_barrier()

    def _out_slice(rbase):
        return out_hbm.at[pl.ds(rbase, FR), pl.ds(cid * HALF, HALF)]

    @pl.loop(sid, N_FCHUNK, step=NS)
    def _fin_chunk(j):
        rbase = pl.multiple_of(j * FR, 8)
        pltpu.async_copy(num_sp.at[pl.ds(rbase, FR)], kb0, semq0)
        pltpu.async_copy(den_sp.at[pl.ds(rbase, FR)], kb1, semk0)
        pltpu.make_async_copy(num_sp.at[pl.ds(rbase, FR)], kb0, semq0).wait()
        pltpu.make_async_copy(den_sp.at[pl.ds(rbase, FR)], kb1, semk0).wait()

        @pl.when(j > sid)
        def _():
            pltpu.make_async_copy(wv0, _out_slice(rbase), semn0).wait()

        @plsc.parallel_loop(0, FR, unroll=2)
        def _fin_row(r):
            for g in range(HALF // LANES):
                sl = pl.ds(g * LANES, LANES)
                num = kb0[r, sl]
                den = kb1[r, sl]
                wv0[r, sl] = jnp.where(den != 0.0, num / den, 0.0)

        pltpu.async_copy(wv0, _out_slice(rbase), semn0)

    pltpu.make_async_copy(wv0, _out_slice(pl.multiple_of(sid * FR, 8)),
                          semn0).wait()


_sc_call = pl.kernel(
    _sc_body,
    out_type=jax.ShapeDtypeStruct((N, G_DIM), jnp.float32),
    mesh=plsc.VectorSubcoreMesh(core_axis_name="c", subcore_axis_name="s",
                                num_cores=NC, num_subcores=NS),
    scratch_types=[
        pltpu.VMEM((2, EC), jnp.int32),
        pltpu.VMEM((2, EC), jnp.int32),
        pltpu.VMEM((EC,), jnp.int32),
        pltpu.VMEM((EC,), jnp.int32),
        pltpu.VMEM((EC,), jnp.int32),
        pltpu.VMEM((EC,), jnp.int32),
        pltpu.VMEM((EC,), jnp.int32),
        pltpu.VMEM((EC,), jnp.int32),
        pltpu.VMEM((EC,), jnp.int32),
        pltpu.VMEM((EC, G_DIM), jnp.float32),
        pltpu.VMEM((EC, G_DIM), jnp.float32),
        pltpu.VMEM((EC, HALF), jnp.float32),
        pltpu.VMEM((EC, HALF), jnp.float32),
        pltpu.VMEM((EC, HALF), jnp.float32),
        pltpu.VMEM((EC, HALF), jnp.float32),
        pltpu.VMEM((EC, HALF), jnp.float32),
        pltpu.VMEM_SHARED((N, HALF), jnp.float32),
        pltpu.VMEM_SHARED((N, HALF), jnp.float32),
        pltpu.SemaphoreType.DMA,
        pltpu.SemaphoreType.DMA,
        pltpu.SemaphoreType.DMA,
        pltpu.SemaphoreType.DMA,
        pltpu.SemaphoreType.DMA,
        pltpu.SemaphoreType.DMA,
        pltpu.SemaphoreType.DMA,
        pltpu.SemaphoreType.DMA,
        pltpu.SemaphoreType.DMA,
        pltpu.SemaphoreType.DMA,
        pltpu.SemaphoreType.DMA,
        pltpu.SemaphoreType.DMA,
        pltpu.SemaphoreType.DMA,
        pltpu.SemaphoreType.DMA,
    ],
    compiler_params=pltpu.CompilerParams(use_tc_tiling_on_sc=False),
)


def kernel(s, edge_index, W_qkv, ln_gamma, ln_beta):
    qv3, k3 = _project(s, W_qkv, ln_gamma, ln_beta)
    qv = qv3.reshape(2 * N, G_DIM)
    kt = k3.reshape(2 * N, HALF)
    ei = edge_index.reshape(2, N_ECHUNK, EC).transpose(1, 0, 2)
    return _sc_call(qv, kt, ei)
